# baseline (speedup 1.0000x reference)
import functools

import jax
import jax.numpy as jnp
from jax import lax
from jax.experimental import pallas as pl
from jax.experimental.pallas import tpu as pltpu
from jax.experimental.pallas import tpu_sc as plsc

V = 1000000
E = 32
B = 16384

_NC = 2
_NS = 16
_NW = _NC * _NS
_CHUNK = B // _NW
_IDXC = 128
_NCHUNK = _CHUNK // _IDXC
_L = 16
_NGROUP = _CHUNK // _L


def _glove_body(center_hbm, outside_hbm, coocs_hbm, wt_hbm,
                cemb_hbm, oemb_hbm, cbias_hbm, obias_hbm, out_hbm,
                idx_c, idx_o, buf_c, buf_o, flat_c, flat_o,
                bias_c, bias_o, cooc_v, wt_v, acc_v, sem):
    wid = lax.axis_index("s") * _NC + lax.axis_index("c")

    pltpu.sync_copy(center_hbm.at[wid], idx_c)
    pltpu.sync_copy(outside_hbm.at[wid], idx_o)
    pltpu.sync_copy(coocs_hbm.at[wid], cooc_v)
    pltpu.sync_copy(wt_hbm.at[wid], wt_v)

    bcopies = []
    for j in range(_NCHUNK):
        sl = pl.ds(j * _IDXC, _IDXC)
        bcopies.append(pltpu.async_copy(cbias_hbm.at[idx_c.at[j]],
                                        bias_c.at[sl], sem))
        bcopies.append(pltpu.async_copy(obias_hbm.at[idx_o.at[j]],
                                        bias_o.at[sl], sem))

    for j in range(_NCHUNK):
        cc = pltpu.async_copy(cemb_hbm.at[idx_c.at[j]], buf_c, sem)
        oc = pltpu.async_copy(oemb_hbm.at[idx_o.at[j]], buf_o, sem)
        cc.wait()
        oc.wait()

        def repack(b, _, j=j):
            dst = j * (_IDXC * E) + b * E
            flat_c[pl.ds(dst, _L)] = buf_c[b, pl.ds(0, _L)]
            flat_c[pl.ds(dst + _L, _L)] = buf_c[b, pl.ds(_L, _L)]
            flat_o[pl.ds(dst, _L)] = buf_o[b, pl.ds(0, _L)]
            flat_o[pl.ds(dst + _L, _L)] = buf_o[b, pl.ds(_L, _L)]
            return 0

        lax.fori_loop(0, _IDXC, repack, 0)

    for c in bcopies:
        c.wait()

    lane_off = lax.iota(jnp.int32, _L) * E

    def body(g, acc):
        base = g * (_L * E)
        idx0 = base + lane_off
        ip = jnp.zeros((_L,), jnp.float32)
        for e in range(E):
            cv = plsc.load_gather(flat_c, [idx0 + e])
            ov = plsc.load_gather(flat_o, [idx0 + e])
            ip = ip + cv * ov
        sl = pl.ds(g * _L, _L)
        d = ip + bias_c[sl] + bias_o[sl] - cooc_v[sl]
        return acc + wt_v[sl] * d * d

    acc = lax.fori_loop(0, _NGROUP, body, jnp.zeros((_L,), jnp.float32))
    acc_v[...] = acc
    pltpu.sync_copy(acc_v, out_hbm.at[wid])


@jax.jit
def _glove(center, outside, coocs, weighting,
           center_emb, outside_emb, center_bias, outside_bias):
    kern = functools.partial(
        pl.kernel,
        mesh=plsc.VectorSubcoreMesh(core_axis_name="c", subcore_axis_name="s"),
        out_type=jax.ShapeDtypeStruct((_NW, _L), jnp.float32),
        compiler_params=pltpu.CompilerParams(needs_layout_passes=False,
                                             use_tc_tiling_on_sc=False),
        scratch_types=[
            pltpu.VMEM((_NCHUNK, _IDXC), jnp.int32),
            pltpu.VMEM((_NCHUNK, _IDXC), jnp.int32),
            pltpu.VMEM((_IDXC, E), jnp.float32),
            pltpu.VMEM((_IDXC, E), jnp.float32),
            pltpu.VMEM((_CHUNK * E,), jnp.float32),
            pltpu.VMEM((_CHUNK * E,), jnp.float32),
            pltpu.VMEM((_CHUNK,), jnp.float32),
            pltpu.VMEM((_CHUNK,), jnp.float32),
            pltpu.VMEM((_CHUNK,), jnp.float32),
            pltpu.VMEM((_CHUNK,), jnp.float32),
            pltpu.VMEM((_L,), jnp.float32),
            pltpu.SemaphoreType.DMA,
        ],
    )(_glove_body)
    partials = kern(center, outside, coocs, weighting,
                    center_emb, outside_emb, center_bias, outside_bias)
    return jnp.sum(partials)


def kernel(center, outside, coocs, weighting,
           center_emb, outside_emb, center_bias, outside_bias):
    center = center.reshape(_NW, _NCHUNK, _IDXC).astype(jnp.int32)
    outside = outside.reshape(_NW, _NCHUNK, _IDXC).astype(jnp.int32)
    coocs = coocs.reshape(_NW, _CHUNK)
    weighting = weighting.reshape(_NW, _CHUNK)
    center_bias = center_bias.reshape(V)
    outside_bias = outside_bias.reshape(V)
    return _glove(center, outside, coocs, weighting,
                  center_emb, outside_emb, center_bias, outside_bias)
